# baseline (speedup 1.0000x reference)
import functools

import numpy as np
import jax
from jax import lax
import jax.numpy as jnp
from jax.experimental import pallas as pl
from jax.experimental.pallas import tpu as pltpu
from jax.experimental.pallas import tpu_sc as plsc


def _line_constants(num_rows: int):
    rng = np.random.default_rng(0)
    rand_row = int(rng.integers(0, num_rows))
    x_min, alpha = 1.0728769e-07, 1.0868737
    r = float(rng.random())
    relative_error = x_min * (1.0 - r) ** (-1.0 / (alpha - 1.0))
    return rand_row, relative_error


_BLOCK_ROWS = 2048


def _sc_scaled_row(forward_input, rand_row, rel_err, n_cols):
    mesh = plsc.VectorSubcoreMesh(core_axis_name="c", subcore_axis_name="s")

    @functools.partial(
        pl.kernel,
        out_type=jax.ShapeDtypeStruct((1, n_cols), forward_input.dtype),
        mesh=mesh,
        scratch_types=[
            pltpu.VMEM((1, n_cols), forward_input.dtype),
            pltpu.SemaphoreType.DMA,
            pltpu.SemaphoreType.DMA,
        ],
    )
    def sc_kernel(x_hbm, row_hbm, buf, rsem, wsem):
        wid = lax.axis_index("s") * 2 + lax.axis_index("c")

        @pl.when(wid == 0)
        def _():
            rd = pltpu.make_async_copy(
                x_hbm.at[pl.ds(rand_row, 1)], buf, rsem
            )
            rd.start()
            rd.wait()
            for k in range(n_cols // 16):
                sl = pl.ds(k * 16, 16)
                buf[0, sl] = buf[0, sl] * jnp.float32(rel_err)
            wr = pltpu.make_async_copy(buf, row_hbm, wsem)
            wr.start()
            wr.wait()

    return sc_kernel(forward_input)


def kernel(forward_input):
    n_rows, n_cols = forward_input.shape
    rand_row, rel_err = _line_constants(n_rows)

    scaled_row = _sc_scaled_row(forward_input, rand_row, rel_err, n_cols)

    block_rows = _BLOCK_ROWS
    grid = n_rows // block_rows
    target_block = rand_row // block_rows
    row_off = rand_row % block_rows

    def body(x_ref, row_ref, o_ref):
        i = pl.program_id(0)
        o_ref[...] = x_ref[...]

        @pl.when(i == target_block)
        def _():
            o_ref[row_off, :] = row_ref[0, :]

    return pl.pallas_call(
        body,
        grid=(grid,),
        in_specs=[
            pl.BlockSpec((block_rows, n_cols), lambda i: (i, 0)),
            pl.BlockSpec((1, n_cols), lambda i: (0, 0)),
        ],
        out_specs=pl.BlockSpec((block_rows, n_cols), lambda i: (i, 0)),
        out_shape=jax.ShapeDtypeStruct((n_rows, n_cols), forward_input.dtype),
    )(forward_input, scaled_row)


# speedup vs baseline: 1.4666x; 1.4666x over previous
import numpy as np
import jax
import jax.numpy as jnp
from jax.experimental import pallas as pl
from jax.experimental.pallas import tpu as pltpu


def _line_constants(num_rows: int):
    rng = np.random.default_rng(0)
    rand_row = int(rng.integers(0, num_rows))
    x_min, alpha = 1.0728769e-07, 1.0868737
    r = float(rng.random())
    relative_error = x_min * (1.0 - r) ** (-1.0 / (alpha - 1.0))
    return rand_row, relative_error


_CHUNK_ROWS = 512
_NBUF = 8
_AHEAD = 4


def kernel(forward_input):
    n_rows, n_cols = forward_input.shape
    rand_row, rel_err = _line_constants(n_rows)

    n_chunks = n_rows // _CHUNK_ROWS
    target_chunk = rand_row // _CHUNK_ROWS
    row_off = rand_row % _CHUNK_ROWS

    def body(x_hbm, o_hbm, bufs, rsems, wsems):
        def read(i):
            return pltpu.make_async_copy(
                x_hbm.at[pl.ds(i * _CHUNK_ROWS, _CHUNK_ROWS)],
                bufs.at[i % _NBUF],
                rsems.at[i % _NBUF],
            )

        def write(i):
            return pltpu.make_async_copy(
                bufs.at[i % _NBUF],
                o_hbm.at[pl.ds(i * _CHUNK_ROWS, _CHUNK_ROWS)],
                wsems.at[i % _NBUF],
            )

        reads = [None] * n_chunks
        writes = [None] * n_chunks
        for i in range(min(_AHEAD, n_chunks)):
            reads[i] = read(i)
            reads[i].start()
        for i in range(n_chunks):
            r = i + _AHEAD
            if r < n_chunks:
                if r >= _NBUF:
                    writes[r - _NBUF].wait()
                reads[r] = read(r)
                reads[r].start()
            reads[i].wait()
            if i == target_chunk:
                b = i % _NBUF
                bufs[b, row_off, :] = bufs[b, row_off, :] * jnp.float32(rel_err)
            writes[i] = write(i)
            writes[i].start()
        for i in range(max(n_chunks - _NBUF, 0), n_chunks):
            writes[i].wait()

    return pl.pallas_call(
        body,
        in_specs=[pl.BlockSpec(memory_space=pl.ANY)],
        out_specs=pl.BlockSpec(memory_space=pl.ANY),
        out_shape=jax.ShapeDtypeStruct((n_rows, n_cols), forward_input.dtype),
        scratch_shapes=[
            pltpu.VMEM((_NBUF, _CHUNK_ROWS, n_cols), forward_input.dtype),
            pltpu.SemaphoreType.DMA((_NBUF,)),
            pltpu.SemaphoreType.DMA((_NBUF,)),
        ],
    )(forward_input)


# speedup vs baseline: 1.4688x; 1.0015x over previous
import numpy as np
import jax
import jax.numpy as jnp
from jax.experimental import pallas as pl
from jax.experimental.pallas import tpu as pltpu


def _line_constants(num_rows: int):
    rng = np.random.default_rng(0)
    rand_row = int(rng.integers(0, num_rows))
    x_min, alpha = 1.0728769e-07, 1.0868737
    r = float(rng.random())
    relative_error = x_min * (1.0 - r) ** (-1.0 / (alpha - 1.0))
    return rand_row, relative_error


_CHUNK_ROWS = 1024
_NBUF = 10
_AHEAD = 5


def kernel(forward_input):
    n_rows, n_cols = forward_input.shape
    rand_row, rel_err = _line_constants(n_rows)

    n_chunks = n_rows // _CHUNK_ROWS
    target_chunk = rand_row // _CHUNK_ROWS
    row_off = rand_row % _CHUNK_ROWS

    def body(x_hbm, o_hbm, bufs, rsems, wsems):
        def read(i):
            return pltpu.make_async_copy(
                x_hbm.at[pl.ds(i * _CHUNK_ROWS, _CHUNK_ROWS)],
                bufs.at[i % _NBUF],
                rsems.at[i % _NBUF],
            )

        def write(i):
            return pltpu.make_async_copy(
                bufs.at[i % _NBUF],
                o_hbm.at[pl.ds(i * _CHUNK_ROWS, _CHUNK_ROWS)],
                wsems.at[i % _NBUF],
            )

        reads = [None] * n_chunks
        writes = [None] * n_chunks
        for i in range(min(_AHEAD, n_chunks)):
            reads[i] = read(i)
            reads[i].start()
        for i in range(n_chunks):
            r = i + _AHEAD
            if r < n_chunks:
                if r >= _NBUF:
                    writes[r - _NBUF].wait()
                reads[r] = read(r)
                reads[r].start()
            reads[i].wait()
            if i == target_chunk:
                b = i % _NBUF
                bufs[b, row_off, :] = bufs[b, row_off, :] * jnp.float32(rel_err)
            writes[i] = write(i)
            writes[i].start()
        for i in range(max(n_chunks - _NBUF, 0), n_chunks):
            writes[i].wait()

    return pl.pallas_call(
        body,
        in_specs=[pl.BlockSpec(memory_space=pl.ANY)],
        out_specs=pl.BlockSpec(memory_space=pl.ANY),
        out_shape=jax.ShapeDtypeStruct((n_rows, n_cols), forward_input.dtype),
        scratch_shapes=[
            pltpu.VMEM((_NBUF, _CHUNK_ROWS, n_cols), forward_input.dtype),
            pltpu.SemaphoreType.DMA((_NBUF,)),
            pltpu.SemaphoreType.DMA((_NBUF,)),
        ],
    )(forward_input)


# speedup vs baseline: 1.4776x; 1.0060x over previous
import numpy as np
import jax
import jax.numpy as jnp
from jax.experimental import pallas as pl


def _line_constants(num_rows: int):
    rng = np.random.default_rng(0)
    rand_row = int(rng.integers(0, num_rows))
    x_min, alpha = 1.0728769e-07, 1.0868737
    r = float(rng.random())
    relative_error = x_min * (1.0 - r) ** (-1.0 / (alpha - 1.0))
    return rand_row, relative_error


_BLOCK_ROWS = 2048


def kernel(forward_input):
    n_rows, n_cols = forward_input.shape
    rand_row, rel_err = _line_constants(n_rows)

    block_rows = _BLOCK_ROWS
    grid = n_rows // block_rows
    target_block = rand_row // block_rows
    row_off = rand_row % block_rows

    def body(x_ref, o_ref):
        i = pl.program_id(0)
        o_ref[...] = x_ref[...]

        @pl.when(i == target_block)
        def _():
            o_ref[row_off, :] = x_ref[row_off, :] * jnp.float32(rel_err)

    return pl.pallas_call(
        body,
        grid=(grid,),
        in_specs=[pl.BlockSpec((block_rows, n_cols), lambda i: (i, 0))],
        out_specs=pl.BlockSpec((block_rows, n_cols), lambda i: (i, 0)),
        out_shape=jax.ShapeDtypeStruct((n_rows, n_cols), forward_input.dtype),
    )(forward_input)
